# baseline (speedup 1.0000x reference)
import functools

import jax
import jax.numpy as jnp
from jax.experimental import pallas as pl
from jax.experimental.pallas import tpu as pltpu

_TB = 256


def _dense_body(idx_ref, x_ref, w_ref, b_ref, o_ref):
    x = x_ref[...]
    idx = idx_ref[0]
    E = w_ref.shape[0]
    K = idx.shape[-1]
    accs = [jnp.zeros(x.shape, jnp.float32) for _ in range(K)]
    for e in range(E):
        y = jax.lax.dot_general(x, w_ref[e], (((1,), (1,)), ((), ())),
                                preferred_element_type=jnp.float32)
        y = y + b_ref[e]
        for k in range(K):
            m = (idx[:, k] == e)[:, None]
            accs[k] = jnp.where(m, y, accs[k])
    for k in range(K):
        o_ref[0, :, k, :] = accs[k]


def kernel(inputs, expert_indices, W, b):
    B, S, D = inputs.shape
    K = expert_indices.shape[-1]
    E = W.shape[0]
    T = B * S
    nb = T // _TB

    flat = inputs.reshape(T, D).astype(jnp.bfloat16)
    idx = expert_indices.astype(jnp.int32).reshape(nb, _TB, K)
    b3 = b.reshape(E, 1, D)
    W = W.astype(jnp.bfloat16)

    out = pl.pallas_call(
        _dense_body,
        grid=(nb,),
        in_specs=[
            pl.BlockSpec((1, _TB, K), lambda i: (i, 0, 0)),
            pl.BlockSpec((_TB, D), lambda i: (i, 0)),
            pl.BlockSpec((E, D, D), lambda i: (0, 0, 0)),
            pl.BlockSpec((E, 1, D), lambda i: (0, 0, 0)),
        ],
        out_specs=pl.BlockSpec((1, _TB, K, D), lambda i: (i, 0, 0, 0)),
        out_shape=jax.ShapeDtypeStruct((nb, _TB, K, D), jnp.float32),
        compiler_params=pltpu.CompilerParams(
            dimension_semantics=("arbitrary",),
        ),
    )(idx, flat, W, b3)
    return out.reshape(B, S, K, D)


# speedup vs baseline: 1.1846x; 1.1846x over previous
import functools

import jax
import jax.numpy as jnp
from jax.experimental import pallas as pl
from jax.experimental.pallas import tpu as pltpu

_TB = 256


def _dense_body(idx_ref, x_ref, w_ref, b_ref, o_ref):
    x = x_ref[...].astype(jnp.bfloat16)
    idx = idx_ref[0]
    E = w_ref.shape[0]
    K = idx.shape[-1]
    accs = [jnp.zeros((x.shape[0], x.shape[1]), jnp.float32) for _ in range(K)]
    for e in range(E):
        y = jax.lax.dot_general(x, w_ref[e].astype(jnp.bfloat16),
                                (((1,), (1,)), ((), ())),
                                preferred_element_type=jnp.float32)
        y = y + b_ref[e]
        for k in range(K):
            m = (idx[:, k] == e)[:, None]
            accs[k] = jnp.where(m, y, accs[k])
    for k in range(K):
        o_ref[0, :, k, :] = accs[k]


def kernel(inputs, expert_indices, W, b):
    B, S, D = inputs.shape
    K = expert_indices.shape[-1]
    E = W.shape[0]
    T = B * S
    nb = T // _TB

    flat = inputs.reshape(T, D)
    idx = expert_indices.astype(jnp.int32).reshape(nb, _TB, K)
    b3 = b.reshape(E, 1, D)

    out = pl.pallas_call(
        _dense_body,
        grid=(nb,),
        in_specs=[
            pl.BlockSpec((1, _TB, K), lambda i: (i, 0, 0)),
            pl.BlockSpec((_TB, D), lambda i: (i, 0)),
            pl.BlockSpec((E, D, D), lambda i: (0, 0, 0)),
            pl.BlockSpec((E, 1, D), lambda i: (0, 0, 0)),
        ],
        out_specs=pl.BlockSpec((1, _TB, K, D), lambda i: (i, 0, 0, 0)),
        out_shape=jax.ShapeDtypeStruct((nb, _TB, K, D), jnp.float32),
        compiler_params=pltpu.CompilerParams(
            dimension_semantics=("arbitrary",),
        ),
    )(idx, flat, W, b3)
    return out.reshape(B, S, K, D)


# speedup vs baseline: 1.2181x; 1.0283x over previous
import functools

import jax
import jax.numpy as jnp
from jax.experimental import pallas as pl
from jax.experimental.pallas import tpu as pltpu

_TB = 512


def _dense_body(idx_ref, x_ref, w_ref, b_ref, o_ref):
    x = x_ref[...].astype(jnp.bfloat16)
    idx = idx_ref[0]
    E = w_ref.shape[0]
    K = idx.shape[-1]
    accs = [jnp.zeros((x.shape[0], x.shape[1]), jnp.float32) for _ in range(K)]
    for e in range(E):
        y = jax.lax.dot_general(x, w_ref[e].astype(jnp.bfloat16),
                                (((1,), (1,)), ((), ())),
                                preferred_element_type=jnp.float32)
        y = y + b_ref[e]
        for k in range(K):
            m = (idx[:, k] == e)[:, None]
            accs[k] = jnp.where(m, y, accs[k])
    for k in range(K):
        o_ref[0, :, k, :] = accs[k]


def kernel(inputs, expert_indices, W, b):
    B, S, D = inputs.shape
    K = expert_indices.shape[-1]
    E = W.shape[0]
    T = B * S
    nb = T // _TB

    flat = inputs.reshape(T, D)
    idx = expert_indices.astype(jnp.int32).reshape(nb, _TB, K)
    b3 = b.reshape(E, 1, D)

    out = pl.pallas_call(
        _dense_body,
        grid=(nb,),
        in_specs=[
            pl.BlockSpec((1, _TB, K), lambda i: (i, 0, 0)),
            pl.BlockSpec((_TB, D), lambda i: (i, 0)),
            pl.BlockSpec((E, D, D), lambda i: (0, 0, 0)),
            pl.BlockSpec((E, 1, D), lambda i: (0, 0, 0)),
        ],
        out_specs=pl.BlockSpec((1, _TB, K, D), lambda i: (i, 0, 0, 0)),
        out_shape=jax.ShapeDtypeStruct((nb, _TB, K, D), jnp.float32),
        compiler_params=pltpu.CompilerParams(
            dimension_semantics=("arbitrary",),
        ),
    )(idx, flat, W, b3)
    return out.reshape(B, S, K, D)
